# baseline (speedup 1.0000x reference)
import functools

import jax
import jax.numpy as jnp
from jax import lax
from jax.experimental import pallas as pl
from jax.experimental.pallas import tpu as pltpu
from jax.experimental.pallas import tpu_sc as plsc

NC = 2
NS = 16
NW = NC * NS
CH = 128


def _sc_mesh():
    return plsc.VectorSubcoreMesh(core_axis_name="c", subcore_axis_name="s",
                                  num_cores=NC, num_subcores=NS)


@functools.lru_cache(maxsize=None)
def _deg_kernel(C: int, SH: int):
    rpt = SH // NS

    def body(dsts, zer1d, out, idx_d, hist_v, acc_v, tmp_v, hist_sh):
        c = lax.axis_index("c")
        s = lax.axis_index("s")
        wid = c * NS + s
        pltpu.sync_copy(dsts.at[wid], idx_d)
        pltpu.sync_copy(zer1d, hist_v)
        ones = jnp.ones((16,), jnp.float32)

        def step(j, carry):
            for l in range(CH // 16):
                idx16 = idx_d[j, pl.ds(l * 16, 16)]
                plsc.addupdate_scatter(hist_v, [idx16], ones)
            return carry

        lax.fori_loop(0, C, step, 0)
        pltpu.sync_copy(hist_v, hist_sh.at[s])
        plsc.subcore_barrier()
        pltpu.sync_copy(hist_sh.at[0, pl.ds(s * rpt, rpt)], acc_v)
        for k in range(1, NS):
            pltpu.sync_copy(hist_sh.at[k, pl.ds(s * rpt, rpt)], tmp_v)

            def addchunk(m, carry):
                sl = pl.ds(m * 16, 16)
                acc_v[sl] = acc_v[sl] + tmp_v[sl]
                return carry

            lax.fori_loop(0, rpt // 16, addchunk, 0)
        pltpu.sync_copy(acc_v, out.at[pl.ds(c * SH + s * rpt, rpt)])

    return pl.kernel(
        body,
        out_type=jax.ShapeDtypeStruct((NC * SH,), jnp.float32),
        mesh=_sc_mesh(),
        scratch_types=[
            pltpu.VMEM((C, CH), jnp.int32),
            pltpu.VMEM((SH,), jnp.float32),
            pltpu.VMEM((rpt,), jnp.float32),
            pltpu.VMEM((rpt,), jnp.float32),
            pltpu.VMEM_SHARED((NS, SH), jnp.float32),
        ],
        compiler_params=pltpu.CompilerParams(needs_layout_passes=False),
    )


NBUF = 2
NHALF = 2


@functools.lru_cache(maxsize=None)
def _agg_kernel(N: int, D: int, C: int, SH: int):
    rpt = SH // NS
    assert C % (NBUF * NHALF) == 0
    C2 = C // NHALF

    def body(hs, srcs, dsts, zer_hbm, out, idx_s, idx_d, *bufs):
        rows = bufs[:NBUF]
        agg_sh = bufs[NBUF]
        ssem = bufs[NBUF + 1:]
        c = lax.axis_index("c")
        s = lax.axis_index("s")
        wid = c * NS + s
        pltpu.sync_copy(zer_hbm, agg_sh.at[pl.ds(s * rpt, rpt)])
        plsc.subcore_barrier()
        for h in range(NHALF):
            pltpu.sync_copy(srcs.at[wid, pl.ds(h * C2, C2)], idx_s)
            pltpu.sync_copy(dsts.at[wid, pl.ds(h * C2, C2)], idx_d)
            for b in range(NBUF):
                pltpu.sync_copy(hs.at[idx_s.at[b]], rows[b])
                pltpu.async_copy(rows[b], agg_sh.at[idx_d.at[b]], ssem[b],
                                 add=True)

            @pl.loop(NBUF, C2, step=NBUF)
            def _rounds(g):
                for b in range(NBUF):
                    j = g + b
                    pltpu.make_async_copy(rows[b], agg_sh.at[idx_d.at[j - NBUF]],
                                          ssem[b]).wait()
                    pltpu.sync_copy(hs.at[idx_s.at[j]], rows[b])
                    pltpu.async_copy(rows[b], agg_sh.at[idx_d.at[j]], ssem[b],
                                     add=True)

            for b in range(NBUF):
                pltpu.make_async_copy(rows[b], agg_sh.at[idx_d.at[C2 - NBUF + b]],
                                      ssem[b]).wait()

        plsc.subcore_barrier()
        pltpu.sync_copy(agg_sh.at[pl.ds(s * rpt, rpt)],
                        out.at[c, pl.ds(s * rpt, rpt)])

    return pl.kernel(
        body,
        out_type=jax.ShapeDtypeStruct((NC, SH, D), jnp.float32),
        mesh=_sc_mesh(),
        scratch_types=[
            pltpu.VMEM((C2, CH), jnp.int32),
            pltpu.VMEM((C2, CH), jnp.int32),
        ] + [pltpu.VMEM((CH, D), jnp.float32)] * NBUF + [
            pltpu.VMEM_SHARED((SH, D), jnp.float32),
        ] + [pltpu.SemaphoreType.DMA] * NBUF,
    )


def _dinv(p0, p1):
    return lax.rsqrt(p0[:, 0:1] + p1[:, 0:1] + 1.0)


def _mm_scale_body(p0, p1, x, w, o):
    dinv = _dinv(p0[...], p1[...])
    h = jnp.dot(x[...], w[...], preferred_element_type=jnp.float32)
    o[...] = h * dinv


def _mid_body(p0, p1, hs, q0, q1, b, w, o):
    dinv = _dinv(p0[...], p1[...])
    t = (hs[...] + q0[...] + q1[...]) * dinv + b[...]
    t = jnp.maximum(t, 0.0)
    o[...] = jnp.dot(t, w[...], preferred_element_type=jnp.float32) * dinv


def _fin_body(p0, p1, hs, q0, q1, b, o):
    dinv = _dinv(p0[...], p1[...])
    o[...] = (hs[...] + q0[...] + q1[...]) * dinv + b[...]


def _row_spec(R, W):
    return pl.BlockSpec((R, W), lambda i: (i, 0))


def _full_spec(S):
    return pl.BlockSpec(S, lambda i: tuple(0 for _ in S))


def _tc_call(body, n_row_ins, has_bias, has_w, N, D, R):
    specs = [_row_spec(R, 16), _row_spec(R, 16)]
    specs += [_row_spec(R, D) for _ in range(n_row_ins)]
    if has_bias:
        specs.append(_full_spec((1, D)))
    if has_w:
        specs.append(_full_spec((D, D)))
    return pl.pallas_call(
        body,
        grid=(N // R,),
        in_specs=specs,
        out_specs=_row_spec(R, D),
        out_shape=jax.ShapeDtypeStruct((N, D), jnp.float32),
    )


def kernel(x, edge_index, W1, b1, W2, b2):
    N, D = x.shape
    E = edge_index.shape[1]
    CQ = NBUF * NHALF
    C = CQ * (-(-E // (NW * CH * CQ)))
    Ep = NW * CH * C
    SH = (NS * CH) * (-(-(N + 1) // (NS * CH)))
    rpt = SH // NS
    R = next(r for r in (2048, 2000, 1024, 1000, 512, 500, 200, 8, 1)
             if N % r == 0 and r % 8 == 0)

    src = edge_index[0].astype(jnp.int32)
    dst = edge_index[1].astype(jnp.int32)
    padn = Ep - E
    srcs = jnp.concatenate([src, jnp.zeros((padn,), jnp.int32)]).reshape(NW, C, CH)
    dsts = jnp.concatenate([dst, jnp.full((padn,), N, jnp.int32)]).reshape(NW, C, CH)

    zerD = jnp.zeros((rpt, D), jnp.float32)
    zer1d = jnp.zeros((SH,), jnp.float32)

    deg1d = _deg_kernel(C, SH)(dsts, zer1d).reshape(NC, SH)
    p0 = jnp.pad(deg1d[0, :N, None], ((0, 0), (0, 15)))
    p1 = jnp.pad(deg1d[1, :N, None], ((0, 0), (0, 15)))

    agg = _agg_kernel(N, D, C, SH)
    b1r = b1.reshape(1, D)
    b2r = b2.reshape(1, D)

    hs1 = _tc_call(_mm_scale_body, 1, False, True, N, D, R)(p0, p1, x, W1)
    a1 = agg(hs1, srcs, dsts, zerD)
    hs2 = _tc_call(_mid_body, 3, True, True, N, D, R)(
        p0, p1, hs1, a1[0, :N], a1[1, :N], b1r, W2)
    a2 = agg(hs2, srcs, dsts, zerD)
    out = _tc_call(_fin_body, 3, True, False, N, D, R)(
        p0, p1, hs2, a2[0, :N], a2[1, :N], b2r)
    return out


# speedup vs baseline: 1.3767x; 1.3767x over previous
import functools

import jax
import jax.numpy as jnp
from jax import lax
from jax.experimental import pallas as pl
from jax.experimental.pallas import tpu as pltpu
from jax.experimental.pallas import tpu_sc as plsc

NC = 2
NS = 16
NW = NC * NS
CH = 128


def _sc_mesh():
    return plsc.VectorSubcoreMesh(core_axis_name="c", subcore_axis_name="s",
                                  num_cores=NC, num_subcores=NS)


@functools.lru_cache(maxsize=None)
def _deg_kernel(C: int, SH: int):
    rpt = SH // NS

    def body(dsts, zer1d, out, idx_d, hist_v, acc_v, tmp_v, hist_sh):
        c = lax.axis_index("c")
        s = lax.axis_index("s")
        wid = c * NS + s
        pltpu.sync_copy(dsts.at[wid], idx_d)
        pltpu.sync_copy(zer1d, hist_v)
        ones = jnp.ones((16,), jnp.float32)

        def step(j, carry):
            for l in range(CH // 16):
                idx16 = idx_d[j, pl.ds(l * 16, 16)]
                plsc.addupdate_scatter(hist_v, [idx16], ones)
            return carry

        lax.fori_loop(0, C, step, 0)
        pltpu.sync_copy(hist_v, hist_sh.at[s])
        plsc.subcore_barrier()
        pltpu.sync_copy(hist_sh.at[0, pl.ds(s * rpt, rpt)], acc_v)
        for k in range(1, NS):
            pltpu.sync_copy(hist_sh.at[k, pl.ds(s * rpt, rpt)], tmp_v)

            def addchunk(m, carry):
                sl = pl.ds(m * 16, 16)
                acc_v[sl] = acc_v[sl] + tmp_v[sl]
                return carry

            lax.fori_loop(0, rpt // 16, addchunk, 0)
        pltpu.sync_copy(acc_v, out.at[pl.ds(c * SH + s * rpt, rpt)])

    return pl.kernel(
        body,
        out_type=jax.ShapeDtypeStruct((NC * SH,), jnp.float32),
        mesh=_sc_mesh(),
        scratch_types=[
            pltpu.VMEM((C, CH), jnp.int32),
            pltpu.VMEM((SH,), jnp.float32),
            pltpu.VMEM((rpt,), jnp.float32),
            pltpu.VMEM((rpt,), jnp.float32),
            pltpu.VMEM_SHARED((NS, SH), jnp.float32),
        ],
        compiler_params=pltpu.CompilerParams(needs_layout_passes=False),
    )


@functools.lru_cache(maxsize=None)
def _agg_kernel(N: int, D: int, C: int, SH: int):
    rpt = SH // NS

    def body(hs, srcs, dsts, zer_hbm, out, idx_s, idx_d, rows, agg_sh):
        c = lax.axis_index("c")
        s = lax.axis_index("s")
        wid = c * NS + s
        pltpu.sync_copy(zer_hbm, agg_sh.at[pl.ds(s * rpt, rpt)])
        pltpu.sync_copy(srcs.at[wid], idx_s)
        pltpu.sync_copy(dsts.at[wid], idx_d)
        plsc.subcore_barrier()

        def step(j, carry):
            pltpu.sync_copy(hs.at[idx_s.at[j]], rows)
            pltpu.sync_copy(rows, agg_sh.at[idx_d.at[j]], add=True)
            return carry

        lax.fori_loop(0, C, step, 0)
        plsc.subcore_barrier()
        pltpu.sync_copy(agg_sh.at[pl.ds(s * rpt, rpt)],
                        out.at[c, pl.ds(s * rpt, rpt)])

    return pl.kernel(
        body,
        out_type=jax.ShapeDtypeStruct((NC, SH, D), jnp.float32),
        mesh=_sc_mesh(),
        scratch_types=[
            pltpu.VMEM((C, CH), jnp.int32),
            pltpu.VMEM((C, CH), jnp.int32),
            pltpu.VMEM((CH, D), jnp.float32),
            pltpu.VMEM_SHARED((SH, D), jnp.float32),
        ],
    )


def _dinv(p0, p1):
    return lax.rsqrt(p0[:, 0:1] + p1[:, 0:1] + 1.0)


def _mm_scale_body(p0, p1, x, w, o):
    dinv = _dinv(p0[...], p1[...])
    h = jnp.dot(x[...], w[...], preferred_element_type=jnp.float32)
    o[...] = h * dinv


def _mid_body(p0, p1, hs, q0, q1, b, w, o):
    dinv = _dinv(p0[...], p1[...])
    t = (hs[...] + q0[...] + q1[...]) * dinv + b[...]
    t = jnp.maximum(t, 0.0)
    o[...] = jnp.dot(t, w[...], preferred_element_type=jnp.float32) * dinv


def _fin_body(p0, p1, hs, q0, q1, b, o):
    dinv = _dinv(p0[...], p1[...])
    o[...] = (hs[...] + q0[...] + q1[...]) * dinv + b[...]


def _row_spec(R, W):
    return pl.BlockSpec((R, W), lambda i: (i, 0))


def _full_spec(S):
    return pl.BlockSpec(S, lambda i: tuple(0 for _ in S))


def _tc_call(body, n_row_ins, has_bias, has_w, N, D, R):
    specs = [_row_spec(R, 16), _row_spec(R, 16)]
    specs += [_row_spec(R, D) for _ in range(n_row_ins)]
    if has_bias:
        specs.append(_full_spec((1, D)))
    if has_w:
        specs.append(_full_spec((D, D)))
    return pl.pallas_call(
        body,
        grid=(N // R,),
        in_specs=specs,
        out_specs=_row_spec(R, D),
        out_shape=jax.ShapeDtypeStruct((N, D), jnp.float32),
    )


def kernel(x, edge_index, W1, b1, W2, b2):
    N, D = x.shape
    E = edge_index.shape[1]
    C = -(-E // (NW * CH))
    Ep = NW * CH * C
    SH = (NS * CH) * (-(-(N + 1) // (NS * CH)))
    rpt = SH // NS
    R = next(r for r in (2048, 2000, 1024, 1000, 512, 500, 200, 8, 1)
             if N % r == 0 and r % 8 == 0)

    src = edge_index[0].astype(jnp.int32)
    dst = edge_index[1].astype(jnp.int32)
    padn = Ep - E
    srcs = jnp.concatenate([src, jnp.zeros((padn,), jnp.int32)]).reshape(NW, C, CH)
    dsts = jnp.concatenate([dst, jnp.full((padn,), N, jnp.int32)]).reshape(NW, C, CH)

    zerD = jnp.zeros((rpt, D), jnp.float32)
    zer1d = jnp.zeros((SH,), jnp.float32)

    deg1d = _deg_kernel(C, SH)(dsts, zer1d).reshape(NC, SH)
    p0 = jnp.pad(deg1d[0, :N, None], ((0, 0), (0, 15)))
    p1 = jnp.pad(deg1d[1, :N, None], ((0, 0), (0, 15)))

    agg = _agg_kernel(N, D, C, SH)
    b1r = b1.reshape(1, D)
    b2r = b2.reshape(1, D)

    hs1 = _tc_call(_mm_scale_body, 1, False, True, N, D, R)(p0, p1, x, W1)
    a1 = agg(hs1, srcs, dsts, zerD)
    hs2 = _tc_call(_mid_body, 3, True, True, N, D, R)(
        p0, p1, hs1, a1[0, :N], a1[1, :N], b1r, W2)
    a2 = agg(hs2, srcs, dsts, zerD)
    out = _tc_call(_fin_body, 3, True, False, N, D, R)(
        p0, p1, hs2, a2[0, :N], a2[1, :N], b2r)
    return out
